# Initial kernel scaffold; baseline (speedup 1.0000x reference)
import jax
import jax.numpy as jnp
from jax.experimental import pallas as pl


def kernel(bond_dir, bond_type, is_in_ring, W_bond_dir, W_bond_type, W_is_in_ring):
    raise NotImplementedError("write your pallas kernel here")


# speedup vs baseline: 5.1153x; 5.1153x over previous
import functools

import jax
import jax.numpy as jnp
from jax import lax
from jax.experimental import pallas as pl
from jax.experimental.pallas import tpu as pltpu
from jax.experimental.pallas import tpu_sc as plsc

E = 800000
D = 64
V1, V2, V3 = 8, 23, 3
NTAB = V1 * V2 * V3
B = 128
NB = E // B
NW = 32
NFULL = NB // NW
NREM = NB % NW


def _table_body(w1_ref, w2_ref, w3_ref, out_ref):
    r = lax.broadcasted_iota(jnp.int32, (NTAB, V1), 0) // (V2 * V3)
    c = lax.broadcasted_iota(jnp.int32, (NTAB, V1), 1)
    t = jnp.dot((r == c).astype(jnp.float32), w1_ref[...],
                preferred_element_type=jnp.float32,
                precision=lax.Precision.HIGHEST)
    r = (lax.broadcasted_iota(jnp.int32, (NTAB, V2), 0) % (V2 * V3)) // V3
    c = lax.broadcasted_iota(jnp.int32, (NTAB, V2), 1)
    t = t + jnp.dot((r == c).astype(jnp.float32), w2_ref[...],
                    preferred_element_type=jnp.float32,
                precision=lax.Precision.HIGHEST)
    r = lax.broadcasted_iota(jnp.int32, (NTAB, V3), 0) % V3
    c = lax.broadcasted_iota(jnp.int32, (NTAB, V3), 1)
    t = t + jnp.dot((r == c).astype(jnp.float32), w3_ref[...],
                    preferred_element_type=jnp.float32,
                precision=lax.Precision.HIGHEST)
    out_ref[...] = t


def _build_table(w1, w2, w3):
    return pl.pallas_call(
        _table_body,
        out_shape=jax.ShapeDtypeStruct((NTAB, D), jnp.float32),
    )(w1, w2, w3)


def _sc_lookup(bd, bt, ir, table):
    mesh = plsc.VectorSubcoreMesh(core_axis_name="c", subcore_axis_name="s")

    @functools.partial(
        pl.kernel,
        mesh=mesh,
        out_type=jax.ShapeDtypeStruct((E, D), jnp.float32),
        compiler_params=pltpu.CompilerParams(use_tc_tiling_on_sc=False),
        scratch_types=[
            pltpu.VMEM((B,), jnp.int32),
            pltpu.VMEM((B,), jnp.int32),
            pltpu.VMEM((B,), jnp.int32),
            pltpu.VMEM((B,), jnp.int32),
            pltpu.VMEM((B, D), jnp.float32),
            pltpu.SemaphoreType.DMA,
        ],
    )
    def k(bd_hbm, bt_hbm, ir_hbm, t_hbm, out_hbm,
          bd_v, bt_v, ir_v, cidx_v, rows_v, sem):
        wid = lax.axis_index("s") * 2 + lax.axis_index("c")

        def block(b):
            base = b * B
            pltpu.sync_copy(bd_hbm.at[pl.ds(base, B)], bd_v)
            pltpu.sync_copy(bt_hbm.at[pl.ds(base, B)], bt_v)
            pltpu.sync_copy(ir_hbm.at[pl.ds(base, B)], ir_v)
            for g in range(B // 16):
                s = pl.ds(g * 16, 16)
                cidx_v[s] = bd_v[s] * (V2 * V3) + bt_v[s] * V3 + ir_v[s]
            pltpu.async_copy(t_hbm.at[cidx_v], rows_v, sem).wait()
            pltpu.sync_copy(rows_v, out_hbm.at[pl.ds(base, B)])

        def body(j, carry):
            block(j * NW + wid)
            return carry

        lax.fori_loop(0, NFULL, body, 0)

        @pl.when(wid < NREM)
        def _tail():
            block(NFULL * NW + wid)

    return k(bd, bt, ir, table)


def kernel(bond_dir, bond_type, is_in_ring, W_bond_dir, W_bond_type,
           W_is_in_ring):
    table = _build_table(W_bond_dir, W_bond_type, W_is_in_ring)
    bd = bond_dir.astype(jnp.int32)
    bt = bond_type.astype(jnp.int32)
    ir = is_in_ring.astype(jnp.int32)
    return _sc_lookup(bd, bt, ir, table)


# speedup vs baseline: 6.6767x; 1.3052x over previous
import functools

import jax
import jax.numpy as jnp
from jax import lax
from jax.experimental import pallas as pl
from jax.experimental.pallas import tpu as pltpu
from jax.experimental.pallas import tpu_sc as plsc

E = 800000
D = 64
V1, V2, V3 = 8, 23, 3
NTAB = V1 * V2 * V3
B = 128
NB = E // B
NW = 32
NFULL = NB // NW
NREM = NB % NW


def _table_body(w1_ref, w2_ref, w3_ref, out_ref):
    r = lax.broadcasted_iota(jnp.int32, (NTAB, V1), 0) // (V2 * V3)
    c = lax.broadcasted_iota(jnp.int32, (NTAB, V1), 1)
    t = jnp.dot((r == c).astype(jnp.float32), w1_ref[...],
                preferred_element_type=jnp.float32,
                precision=lax.Precision.HIGHEST)
    r = (lax.broadcasted_iota(jnp.int32, (NTAB, V2), 0) % (V2 * V3)) // V3
    c = lax.broadcasted_iota(jnp.int32, (NTAB, V2), 1)
    t = t + jnp.dot((r == c).astype(jnp.float32), w2_ref[...],
                    preferred_element_type=jnp.float32,
                precision=lax.Precision.HIGHEST)
    r = lax.broadcasted_iota(jnp.int32, (NTAB, V3), 0) % V3
    c = lax.broadcasted_iota(jnp.int32, (NTAB, V3), 1)
    t = t + jnp.dot((r == c).astype(jnp.float32), w3_ref[...],
                    preferred_element_type=jnp.float32,
                precision=lax.Precision.HIGHEST)
    out_ref[...] = t


def _build_table(w1, w2, w3):
    return pl.pallas_call(
        _table_body,
        out_shape=jax.ShapeDtypeStruct((NTAB, D), jnp.float32),
    )(w1, w2, w3)


NBUF = 2
NT = NFULL + (1 if NREM else 0)
NOUT = NT // NBUF


def _sc_lookup(bd, bt, ir, table):
    mesh = plsc.VectorSubcoreMesh(core_axis_name="c", subcore_axis_name="s")

    per_buf = [
        pltpu.VMEM((B,), jnp.int32),
        pltpu.VMEM((B,), jnp.int32),
        pltpu.VMEM((B,), jnp.int32),
        pltpu.VMEM((B,), jnp.int32),
        pltpu.VMEM((B, D), jnp.float32),
        pltpu.SemaphoreType.DMA,
        pltpu.SemaphoreType.DMA,
        pltpu.SemaphoreType.DMA,
    ]

    @functools.partial(
        pl.kernel,
        mesh=mesh,
        out_type=jax.ShapeDtypeStruct((E, D), jnp.float32),
        compiler_params=pltpu.CompilerParams(use_tc_tiling_on_sc=False),
        scratch_types=per_buf * NBUF,
    )
    def k(bd_hbm, bt_hbm, ir_hbm, t_hbm, out_hbm, *scratch):
        bufs = [scratch[i * 8:(i + 1) * 8] for i in range(NBUF)]
        wid = lax.axis_index("s") * 2 + lax.axis_index("c")

        def base_of(j):
            return jnp.minimum(j * NW + wid, NB - 1) * B

        def fire_idx(j, p):
            bd_v, bt_v, ir_v, _, _, si, _, _ = bufs[p]
            base = base_of(j)
            pltpu.async_copy(bd_hbm.at[pl.ds(base, B)], bd_v, si)
            pltpu.async_copy(bt_hbm.at[pl.ds(base, B)], bt_v, si)
            pltpu.async_copy(ir_hbm.at[pl.ds(base, B)], ir_v, si)

        def wait_idx(p):
            bd_v, bt_v, ir_v, _, _, si, _, _ = bufs[p]
            pltpu.make_async_copy(bd_hbm.at[pl.ds(0, B)], bd_v, si).wait()
            pltpu.make_async_copy(bt_hbm.at[pl.ds(0, B)], bt_v, si).wait()
            pltpu.make_async_copy(ir_hbm.at[pl.ds(0, B)], ir_v, si).wait()

        def wait_store(p):
            rows_v, ss = bufs[p][4], bufs[p][7]
            pltpu.make_async_copy(rows_v, out_hbm.at[pl.ds(0, B)], ss).wait()

        for p in range(NBUF):
            fire_idx(p, p)

        def body(i, carry):
            for p in range(NBUF):
                bd_v, bt_v, ir_v, cidx_v, rows_v, si, sg, ss = bufs[p]
                wait_idx(p)
                for g in range(B // 16):
                    s = pl.ds(g * 16, 16)
                    cidx_v[s] = bd_v[s] * (V2 * V3) + bt_v[s] * V3 + ir_v[s]

                @pl.when(i > 0)
                def _drain_store(p=p):
                    wait_store(p)

                pltpu.async_copy(t_hbm.at[cidx_v], rows_v, sg)

            @pl.when(i + 1 < NOUT)
            def _prefetch():
                for p in range(NBUF):
                    fire_idx((i + 1) * NBUF + p, p)

            for p in range(NBUF):
                _, _, _, _, rows_v, si, sg, ss = bufs[p]
                pltpu.make_async_copy(t_hbm.at[pl.ds(0, B)], rows_v, sg).wait()
                pltpu.async_copy(rows_v,
                                 out_hbm.at[pl.ds(base_of(i * NBUF + p), B)],
                                 ss)
            return carry

        lax.fori_loop(0, NOUT, body, 0)

        for p in range(NBUF):
            wait_store(p)

    return k(bd, bt, ir, table)


def kernel(bond_dir, bond_type, is_in_ring, W_bond_dir, W_bond_type,
           W_is_in_ring):
    table = _build_table(W_bond_dir, W_bond_type, W_is_in_ring)
    bd = bond_dir.astype(jnp.int32)
    bt = bond_type.astype(jnp.int32)
    ir = is_in_ring.astype(jnp.int32)
    return _sc_lookup(bd, bt, ir, table)


# speedup vs baseline: 6.7208x; 1.0066x over previous
import functools

import jax
import jax.numpy as jnp
from jax import lax
from jax.experimental import pallas as pl
from jax.experimental.pallas import tpu as pltpu
from jax.experimental.pallas import tpu_sc as plsc

E = 800000
D = 64
V1, V2, V3 = 8, 23, 3
NTAB = V1 * V2 * V3
B = 128
NB = E // B
NW = 32
NFULL = NB // NW
NREM = NB % NW


def _table_body(w1_ref, w2_ref, w3_ref, out_ref):
    r = lax.broadcasted_iota(jnp.int32, (NTAB, V1), 0) // (V2 * V3)
    c = lax.broadcasted_iota(jnp.int32, (NTAB, V1), 1)
    t = jnp.dot((r == c).astype(jnp.float32), w1_ref[...],
                preferred_element_type=jnp.float32,
                precision=lax.Precision.HIGHEST)
    r = (lax.broadcasted_iota(jnp.int32, (NTAB, V2), 0) % (V2 * V3)) // V3
    c = lax.broadcasted_iota(jnp.int32, (NTAB, V2), 1)
    t = t + jnp.dot((r == c).astype(jnp.float32), w2_ref[...],
                    preferred_element_type=jnp.float32,
                precision=lax.Precision.HIGHEST)
    r = lax.broadcasted_iota(jnp.int32, (NTAB, V3), 0) % V3
    c = lax.broadcasted_iota(jnp.int32, (NTAB, V3), 1)
    t = t + jnp.dot((r == c).astype(jnp.float32), w3_ref[...],
                    preferred_element_type=jnp.float32,
                precision=lax.Precision.HIGHEST)
    out_ref[...] = t


def _build_table(w1, w2, w3):
    return pl.pallas_call(
        _table_body,
        out_shape=jax.ShapeDtypeStruct((NTAB, D), jnp.float32),
    )(w1, w2, w3)


NBUF = 4
NT = NFULL + (1 if NREM else 0)
NOUT = NT // NBUF


def _sc_lookup(bd, bt, ir, table):
    mesh = plsc.VectorSubcoreMesh(core_axis_name="c", subcore_axis_name="s")

    per_buf = [
        pltpu.VMEM((B,), jnp.int32),
        pltpu.VMEM((B,), jnp.int32),
        pltpu.VMEM((B,), jnp.int32),
        pltpu.VMEM((B,), jnp.int32),
        pltpu.VMEM((B, D), jnp.float32),
        pltpu.SemaphoreType.DMA,
        pltpu.SemaphoreType.DMA,
        pltpu.SemaphoreType.DMA,
    ]

    @functools.partial(
        pl.kernel,
        mesh=mesh,
        out_type=jax.ShapeDtypeStruct((E, D), jnp.float32),
        compiler_params=pltpu.CompilerParams(use_tc_tiling_on_sc=False),
        scratch_types=per_buf * NBUF,
    )
    def k(bd_hbm, bt_hbm, ir_hbm, t_hbm, out_hbm, *scratch):
        bufs = [scratch[i * 8:(i + 1) * 8] for i in range(NBUF)]
        wid = lax.axis_index("s") * 2 + lax.axis_index("c")

        def base_of(j):
            return jnp.minimum(j * NW + wid, NB - 1) * B

        def fire_idx(j, p):
            bd_v, bt_v, ir_v, _, _, si, _, _ = bufs[p]
            base = base_of(j)
            pltpu.async_copy(bd_hbm.at[pl.ds(base, B)], bd_v, si)
            pltpu.async_copy(bt_hbm.at[pl.ds(base, B)], bt_v, si)
            pltpu.async_copy(ir_hbm.at[pl.ds(base, B)], ir_v, si)

        def wait_idx(p):
            bd_v, bt_v, ir_v, _, _, si, _, _ = bufs[p]
            pltpu.make_async_copy(bd_hbm.at[pl.ds(0, B)], bd_v, si).wait()
            pltpu.make_async_copy(bt_hbm.at[pl.ds(0, B)], bt_v, si).wait()
            pltpu.make_async_copy(ir_hbm.at[pl.ds(0, B)], ir_v, si).wait()

        def wait_store(p):
            rows_v, ss = bufs[p][4], bufs[p][7]
            pltpu.make_async_copy(rows_v, out_hbm.at[pl.ds(0, B)], ss).wait()

        for p in range(NBUF):
            fire_idx(p, p)

        def body(i, carry):
            for p in range(NBUF):
                bd_v, bt_v, ir_v, cidx_v, rows_v, si, sg, ss = bufs[p]
                wait_idx(p)
                for g in range(B // 16):
                    s = pl.ds(g * 16, 16)
                    cidx_v[s] = bd_v[s] * (V2 * V3) + bt_v[s] * V3 + ir_v[s]

                @pl.when(i > 0)
                def _drain_store(p=p):
                    wait_store(p)

                pltpu.async_copy(t_hbm.at[cidx_v], rows_v, sg)

            @pl.when(i + 1 < NOUT)
            def _prefetch():
                for p in range(NBUF):
                    fire_idx((i + 1) * NBUF + p, p)

            for p in range(NBUF):
                _, _, _, _, rows_v, si, sg, ss = bufs[p]
                pltpu.make_async_copy(t_hbm.at[pl.ds(0, B)], rows_v, sg).wait()
                pltpu.async_copy(rows_v,
                                 out_hbm.at[pl.ds(base_of(i * NBUF + p), B)],
                                 ss)
            return carry

        lax.fori_loop(0, NOUT, body, 0)

        for p in range(NBUF):
            wait_store(p)

    return k(bd, bt, ir, table)


def kernel(bond_dir, bond_type, is_in_ring, W_bond_dir, W_bond_type,
           W_is_in_ring):
    table = _build_table(W_bond_dir, W_bond_type, W_is_in_ring)
    bd = bond_dir.astype(jnp.int32)
    bt = bond_type.astype(jnp.int32)
    ir = is_in_ring.astype(jnp.int32)
    return _sc_lookup(bd, bt, ir, table)
